# baseline (speedup 1.0000x reference)
import functools

import jax
import jax.numpy as jnp
from jax import lax
from jax.experimental import pallas as pl
from jax.experimental.pallas import tpu as pltpu
from jax.experimental.pallas import tpu_sc as plsc

N = 320000
D = 128
S = 1024
NC = 2
NS = 16
NW = NC * NS
RPW = N // NW
C = 80
NCHUNK = RPW // C
NV = RPW // 16
SEG_PER_TILE = S // NS
NBUF = 3
LR = 2 * C
NPAIR = 62


def _sc_body(x_hbm, ids_hbm, psum_hbm, pcnt_hbm,
             ids_v, b0, b1, b2, zbuf, tmp16,
             startb, endb, cntb, acc, sem_l, sem_s):
    cid = lax.axis_index("c")
    sid = lax.axis_index("s")
    wid = sid * NC + cid
    bufs = (b0, b1, b2)

    zeros16 = jnp.zeros((16,), jnp.float32)
    zeros16i = jnp.zeros((16,), jnp.int32)
    iota16 = lax.iota(jnp.int32, 16)
    lane0 = iota16 == 0
    shift_idx = jnp.maximum(iota16 - 1, 0)
    last_idx = jnp.full((16,), 15, jnp.int32)

    def init_zbuf(i, _):
        for k in range(D // 16):
            zbuf[i, pl.ds(k * 16, 16)] = zeros16
        return 0

    lax.fori_loop(0, SEG_PER_TILE, init_zbuf, 0)

    def init_bounds(i, _):
        startb[pl.ds(i * 16, 16)] = zeros16i
        endb[pl.ds(i * 16, 16)] = zeros16i
        return 0

    lax.fori_loop(0, S // 16, init_bounds, 0)

    pltpu.sync_copy(zbuf, acc.at[pl.ds(sid * SEG_PER_TILE, SEG_PER_TILE)])

    pltpu.sync_copy(ids_hbm.at[wid], ids_v)

    plsc.subcore_barrier()

    row0 = wid * RPW

    def prows(p):
        return x_hbm.at[pl.ds(row0 + p * LR, LR)]

    def load(p, k):
        pltpu.async_copy(prows(p), bufs[k], sem_l)

    def wait_load(p, k):
        pltpu.make_async_copy(prows(p), bufs[k], sem_l).wait()

    def scat(j, k, h):
        pltpu.async_copy(bufs[k].at[pl.ds(h * C, C)],
                         acc.at[ids_v.at[j]], sem_s, add=True)

    def wait_scat(j, k, h):
        pltpu.make_async_copy(bufs[k].at[pl.ds(h * C, C)],
                              acc.at[ids_v.at[j]], sem_s).wait()

    load(0, 0)
    load(1, 1)

    def sweep(j, carry):
        for cb in range(C // 16):
            v = ids_v[j, pl.ds(cb * 16, 16)]
            tmp16[...] = v
            shifted = plsc.load_gather(tmp16, [shift_idx])
            prev = jnp.where(lane0, carry, shifted)
            is_start = v != prev
            pos = j * C + cb * 16 + iota16
            plsc.store_scatter(startb, [v], pos, mask=is_start)
            plsc.store_scatter(endb, [jnp.maximum(prev, 0)], pos,
                               mask=is_start & (prev >= 0))
            carry = plsc.load_gather(tmp16, [last_idx])
        return carry

    last_id = lax.fori_loop(0, NCHUNK, sweep, jnp.full((16,), -1, jnp.int32))
    plsc.store_scatter(endb, [last_id],
                       jnp.full((16,), RPW, jnp.int32), mask=lane0)

    def finalize(i, _):
        s16 = startb[pl.ds(i * 16, 16)]
        e16 = endb[pl.ds(i * 16, 16)]
        cntb[pl.ds(i * 16, 16)] = e16 - s16
        return 0

    lax.fori_loop(0, S // 16, finalize, 0)

    def pair(p, k, first, last):
        wait_load(p, k)
        scat(2 * p, k, 0)
        scat(2 * p + 1, k, 1)
        if not first:
            wait_scat(2 * p - 2, (k - 1) % NBUF, 0)
            wait_scat(2 * p - 1, (k - 1) % NBUF, 1)
        if not last:
            load(p + 2, (k + 2) % NBUF)

    pair(0, 0, True, False)
    pair(1, 1, False, False)
    pair(2, 2, False, False)

    def group(t, _):
        p0 = NBUF * t
        for k in range(NBUF):
            pair(p0 + k, k, False, False)
        return 0

    lax.fori_loop(1, NPAIR // NBUF, group, 0)

    pair(60, 0, False, True)
    pair(61, 1, False, True)
    wait_scat(2 * 61, 1, 0)
    wait_scat(2 * 61 + 1, 1, 1)

    jl = NCHUNK - 1
    pltpu.sync_copy(x_hbm.at[pl.ds(row0 + jl * C, C)], bufs[2].at[pl.ds(0, C)])
    pltpu.sync_copy(bufs[2].at[pl.ds(0, C)], acc.at[ids_v.at[jl]], add=True)

    pltpu.sync_copy(cntb, pcnt_hbm.at[wid])

    plsc.subcore_barrier()

    seg0 = sid * SEG_PER_TILE
    pltpu.sync_copy(acc.at[pl.ds(seg0, SEG_PER_TILE)], zbuf)
    pltpu.sync_copy(zbuf, psum_hbm.at[cid].at[pl.ds(seg0, SEG_PER_TILE)])


_sc_call = functools.partial(
    pl.kernel,
    out_type=[
        jax.ShapeDtypeStruct((NC, S, D), jnp.float32),
        jax.ShapeDtypeStruct((NW, S), jnp.int32),
    ],
    mesh=plsc.VectorSubcoreMesh(core_axis_name="c", subcore_axis_name="s"),
    compiler_params=pltpu.CompilerParams(needs_layout_passes=False),
    scratch_types=[
        pltpu.VMEM((NCHUNK, C), jnp.int32),
        pltpu.VMEM((LR, D), jnp.float32),
        pltpu.VMEM((LR, D), jnp.float32),
        pltpu.VMEM((LR, D), jnp.float32),
        pltpu.VMEM((SEG_PER_TILE, D), jnp.float32),
        pltpu.VMEM((16,), jnp.int32),
        pltpu.VMEM((S,), jnp.int32),
        pltpu.VMEM((S,), jnp.int32),
        pltpu.VMEM((S,), jnp.int32),
        pltpu.VMEM_SHARED((S, D), jnp.float32),
        pltpu.SemaphoreType.DMA,
        pltpu.SemaphoreType.DMA,
    ],
)(_sc_body)


def _combine_body(p_ref, c_ref, o_ref):
    sums = p_ref[0] + p_ref[1]
    cnt = jnp.sum(c_ref[...].astype(jnp.float32), axis=0)
    row_i = lax.broadcasted_iota(jnp.int32, (S, S), 0)
    col_i = lax.broadcasted_iota(jnp.int32, (S, S), 1)
    diag_cnt = jnp.where(row_i == col_i, cnt[None, :], 0.0)
    cnt_rows = jnp.dot(diag_cnt, jnp.ones((S, D), jnp.float32),
                       precision=lax.Precision.HIGHEST,
                       preferred_element_type=jnp.float32)
    o_ref[...] = sums / jnp.maximum(cnt_rows, 1.0)


_combine = pl.pallas_call(
    _combine_body,
    out_shape=jax.ShapeDtypeStruct((S, D), jnp.float32),
)


def kernel(x, batch):
    ids = batch.astype(jnp.int32).reshape(NW, NCHUNK, C)
    psum, pcnt = _sc_call(x, ids)
    return _combine(psum, pcnt)


# speedup vs baseline: 1.0782x; 1.0782x over previous
import functools

import jax
import jax.numpy as jnp
from jax import lax
from jax.experimental import pallas as pl
from jax.experimental.pallas import tpu as pltpu
from jax.experimental.pallas import tpu_sc as plsc

N = 320000
D = 128
S = 1024
NC = 2
NS = 16
NW = NC * NS
RPW = N // NW
C = 80
NCHUNK = RPW // C
NV = RPW // 16
SEG_PER_TILE = S // NS
NBUF = 5


def _sc_body(x_hbm, ids_hbm, psum_hbm, pcnt_hbm,
             ids_v, b0, b1, b2, b3, b4, zbuf, tmp16,
             startb, endb, cntb, acc, sem_l, sem_s):
    cid = lax.axis_index("c")
    sid = lax.axis_index("s")
    wid = sid * NC + cid
    bufs = (b0, b1, b2, b3, b4)

    zeros16 = jnp.zeros((16,), jnp.float32)
    zeros16i = jnp.zeros((16,), jnp.int32)
    iota16 = lax.iota(jnp.int32, 16)
    lane0 = iota16 == 0
    shift_idx = jnp.maximum(iota16 - 1, 0)
    last_idx = jnp.full((16,), 15, jnp.int32)

    def init_zbuf(i, _):
        for k in range(D // 16):
            zbuf[i, pl.ds(k * 16, 16)] = zeros16
        return 0

    lax.fori_loop(0, SEG_PER_TILE, init_zbuf, 0)

    def init_bounds(i, _):
        startb[pl.ds(i * 16, 16)] = zeros16i
        endb[pl.ds(i * 16, 16)] = zeros16i
        return 0

    lax.fori_loop(0, S // 16, init_bounds, 0)

    pltpu.sync_copy(zbuf, acc.at[pl.ds(sid * SEG_PER_TILE, SEG_PER_TILE)])

    pltpu.sync_copy(ids_hbm.at[wid], ids_v)

    plsc.subcore_barrier()

    row0 = wid * RPW

    def rows(j):
        return x_hbm.at[pl.ds(row0 + j * C, C)]

    def load(j, k):
        pltpu.async_copy(rows(j), bufs[k], sem_l)

    def wait_load(j, k):
        pltpu.make_async_copy(rows(j), bufs[k], sem_l).wait()

    def scat(j, k):
        pltpu.async_copy(bufs[k], acc.at[ids_v.at[j]], sem_s, add=True)

    def wait_scat(j, k):
        pltpu.make_async_copy(bufs[k], acc.at[ids_v.at[j]], sem_s).wait()

    load(0, 0)
    load(1, 1)
    load(2, 2)
    load(3, 3)

    def sweep_row(j, carry):
        for cb in range(C // 16):
            v = ids_v[j, pl.ds(cb * 16, 16)]
            tmp16[...] = v
            shifted = plsc.load_gather(tmp16, [shift_idx])
            prev = jnp.where(lane0, carry, shifted)
            is_start = v != prev
            pos = j * C + cb * 16 + iota16
            plsc.store_scatter(startb, [v], pos, mask=is_start)
            plsc.store_scatter(endb, [jnp.maximum(prev, 0)], pos,
                               mask=is_start & (prev >= 0))
            carry = plsc.load_gather(tmp16, [last_idx])
        return carry

    wait_load(0, 0)
    scat(0, 0)
    load(4, 4)
    carry = sweep_row(0, jnp.full((16,), -1, jnp.int32))
    for k in range(1, NBUF):
        wait_load(k, k)
        scat(k, k)
        carry = sweep_row(k, carry)
        wait_scat(k - 1, k - 1)
        load(k + NBUF - 1, (k + NBUF - 1) % NBUF)

    def group(t, carry):
        j0 = NBUF * t
        for k in range(NBUF):
            j = j0 + k
            wait_load(j, k)
            scat(j, k)
            carry = sweep_row(j, carry)
            wait_scat(j - 1, (k - 1) % NBUF)
            load(j + NBUF - 1, (k + NBUF - 1) % NBUF)
        return carry

    carry = lax.fori_loop(1, NCHUNK // NBUF - 1, group, carry)

    j0 = NCHUNK - NBUF
    for k in range(NBUF):
        j = j0 + k
        wait_load(j, k)
        scat(j, k)
        carry = sweep_row(j, carry)
        wait_scat(j - 1, (k - 1) % NBUF)
        if j + NBUF - 1 < NCHUNK:
            load(j + NBUF - 1, (k + NBUF - 1) % NBUF)

    plsc.store_scatter(endb, [carry],
                       jnp.full((16,), RPW, jnp.int32), mask=lane0)

    def finalize(i, _):
        s16 = startb[pl.ds(i * 16, 16)]
        e16 = endb[pl.ds(i * 16, 16)]
        cntb[pl.ds(i * 16, 16)] = e16 - s16
        return 0

    lax.fori_loop(0, S // 16, finalize, 0)
    wait_scat(NCHUNK - 1, (NCHUNK - 1) % NBUF)

    pltpu.sync_copy(cntb, pcnt_hbm.at[wid])

    plsc.subcore_barrier()

    seg0 = sid * SEG_PER_TILE
    pltpu.sync_copy(acc.at[pl.ds(seg0, SEG_PER_TILE)],
                    psum_hbm.at[cid].at[pl.ds(seg0, SEG_PER_TILE)])


_sc_call = functools.partial(
    pl.kernel,
    out_type=[
        jax.ShapeDtypeStruct((NC, S, D), jnp.float32),
        jax.ShapeDtypeStruct((NW, S), jnp.int32),
    ],
    mesh=plsc.VectorSubcoreMesh(core_axis_name="c", subcore_axis_name="s"),
    compiler_params=pltpu.CompilerParams(needs_layout_passes=False),
    scratch_types=[
        pltpu.VMEM((NCHUNK, C), jnp.int32),
        pltpu.VMEM((C, D), jnp.float32),
        pltpu.VMEM((C, D), jnp.float32),
        pltpu.VMEM((C, D), jnp.float32),
        pltpu.VMEM((C, D), jnp.float32),
        pltpu.VMEM((C, D), jnp.float32),
        pltpu.VMEM((SEG_PER_TILE, D), jnp.float32),
        pltpu.VMEM((16,), jnp.int32),
        pltpu.VMEM((S,), jnp.int32),
        pltpu.VMEM((S,), jnp.int32),
        pltpu.VMEM((S,), jnp.int32),
        pltpu.VMEM_SHARED((S, D), jnp.float32),
        pltpu.SemaphoreType.DMA,
        pltpu.SemaphoreType.DMA,
    ],
)(_sc_body)


def _combine_body(p_ref, c_ref, o_ref):
    sums = p_ref[0] + p_ref[1]
    cnt = jnp.sum(c_ref[...].astype(jnp.float32), axis=0)
    row_i = lax.broadcasted_iota(jnp.int32, (S, S), 0)
    col_i = lax.broadcasted_iota(jnp.int32, (S, S), 1)
    diag_cnt = jnp.where(row_i == col_i, cnt[None, :], 0.0)
    cnt_rows = jnp.dot(diag_cnt, jnp.ones((S, D), jnp.float32),
                       precision=lax.Precision.HIGHEST,
                       preferred_element_type=jnp.float32)
    o_ref[...] = sums / jnp.maximum(cnt_rows, 1.0)


_combine = pl.pallas_call(
    _combine_body,
    out_shape=jax.ShapeDtypeStruct((S, D), jnp.float32),
)


def kernel(x, batch):
    ids = batch.astype(jnp.int32).reshape(NW, NCHUNK, C)
    psum, pcnt = _sc_call(x, ids)
    return _combine(psum, pcnt)


# speedup vs baseline: 1.5255x; 1.4149x over previous
import functools

import jax
import jax.numpy as jnp
from jax import lax
from jax.experimental import pallas as pl
from jax.experimental.pallas import tpu as pltpu
from jax.experimental.pallas import tpu_sc as plsc

N = 320000
D = 128
S = 1024
NC = 2
NS = 16
NW = NC * NS
RPW = N // NW
C = 80
NCHUNK = RPW // C
NV = RPW // 16
SEG_PER_TILE = S // NS
NBUF = 5


def _sc_body(x_hbm, ids_hbm, cfid_hbm, psum_hbm, pcnt_hbm,
             ids_v, cfid_v, b0, b1, b2, b3, b4, sbuf, zbuf, tmp16,
             startb, endb, cntb, acc, sem_l, sem_s):
    cid = lax.axis_index("c")
    sid = lax.axis_index("s")
    wid = sid * NC + cid
    bufs = (b0, b1, b2, b3, b4)

    zeros16 = jnp.zeros((16,), jnp.float32)
    zeros16i = jnp.zeros((16,), jnp.int32)
    iota16 = lax.iota(jnp.int32, 16)
    lane0 = iota16 == 0
    shift_idx = jnp.maximum(iota16 - 1, 0)
    last_idx = jnp.full((16,), 15, jnp.int32)

    def init_zbuf(i, _):
        for k in range(D // 16):
            zbuf[i, pl.ds(k * 16, 16)] = zeros16
        return 0

    lax.fori_loop(0, SEG_PER_TILE, init_zbuf, 0)

    def init_sbuf(i, _):
        for k in range(D // 16):
            sbuf[i, pl.ds(k * 16, 16)] = zeros16
        return 0

    lax.fori_loop(0, NBUF * 8, init_sbuf, 0)

    def init_bounds(i, _):
        startb[pl.ds(i * 16, 16)] = zeros16i
        endb[pl.ds(i * 16, 16)] = zeros16i
        return 0

    lax.fori_loop(0, S // 16, init_bounds, 0)

    pltpu.sync_copy(zbuf, acc.at[pl.ds(sid * SEG_PER_TILE, SEG_PER_TILE)])

    pltpu.sync_copy(ids_hbm.at[wid], ids_v)
    pltpu.sync_copy(cfid_hbm.at[wid], cfid_v)

    plsc.subcore_barrier()

    row0 = wid * RPW

    def rows(j):
        return x_hbm.at[pl.ds(row0 + j * C, C)]

    def load(j, k):
        pltpu.async_copy(rows(j), bufs[k], sem_l)

    def wait_load(j, k):
        pltpu.make_async_copy(rows(j), bufs[k], sem_l).wait()

    def scat(j, k):
        pltpu.async_copy(bufs[k], acc.at[ids_v.at[j]], sem_s, add=True)

    def wait_scat(j, k):
        pltpu.make_async_copy(bufs[k], acc.at[ids_v.at[j]], sem_s).wait()

    def scat1(j, k):
        pltpu.async_copy(sbuf.at[pl.ds(8 * k, 8)],
                         acc.at[cfid_v.at[j]], sem_s, add=True)

    def wait_scat1(j, k):
        pltpu.make_async_copy(sbuf.at[pl.ds(8 * k, 8)],
                              acc.at[cfid_v.at[j]], sem_s).wait()

    def uniform(j):
        v0 = ids_v[j, pl.ds(0, 16)]
        v4 = ids_v[j, pl.ds(C - 16, 16)]
        return v0[0] == v4[15]

    def cond_wait_scat(j, k):
        pred = uniform(j)

        @pl.when(pred)
        def _():
            wait_scat1(j, k)

        @pl.when(jnp.logical_not(pred))
        def _():
            wait_scat(j, k)

    def do_chunk(j, k):
        pred = uniform(j)

        @pl.when(pred)
        def _():
            bk = bufs[k]

            def dorow(r, accs):
                return tuple(accs[m] + bk[r, pl.ds(16 * m, 16)]
                             for m in range(D // 16))

            accs = lax.fori_loop(
                0, C, dorow,
                tuple(zeros16 for _ in range(D // 16)))
            for m in range(D // 16):
                sbuf[8 * k, pl.ds(16 * m, 16)] = accs[m]
            scat1(j, k)

        @pl.when(jnp.logical_not(pred))
        def _():
            scat(j, k)

    load(0, 0)
    load(1, 1)
    load(2, 2)

    def sweep_row(j, carry):
        for cb in range(C // 16):
            v = ids_v[j, pl.ds(cb * 16, 16)]
            tmp16[...] = v
            shifted = plsc.load_gather(tmp16, [shift_idx])
            prev = jnp.where(lane0, carry, shifted)
            is_start = v != prev
            pos = j * C + cb * 16 + iota16
            plsc.store_scatter(startb, [v], pos, mask=is_start)
            plsc.store_scatter(endb, [jnp.maximum(prev, 0)], pos,
                               mask=is_start & (prev >= 0))
            carry = plsc.load_gather(tmp16, [last_idx])
        return carry

    carry = jnp.full((16,), -1, jnp.int32)
    for k in range(NBUF):
        j = k
        wait_load(j, k)
        if j >= 2:
            cond_wait_scat(j - 2, (k - 2) % NBUF)
        do_chunk(j, k)
        carry = sweep_row(j, carry)
        load(j + 3, (k + 3) % NBUF)

    def group(t, carry):
        j0 = NBUF * t
        for k in range(NBUF):
            j = j0 + k
            wait_load(j, k)
            cond_wait_scat(j - 2, (k - 2) % NBUF)
            do_chunk(j, k)
            carry = sweep_row(j, carry)
            load(j + 3, (k + 3) % NBUF)
        return carry

    carry = lax.fori_loop(1, NCHUNK // NBUF - 1, group, carry)

    j0 = NCHUNK - NBUF
    for k in range(NBUF):
        j = j0 + k
        wait_load(j, k)
        cond_wait_scat(j - 2, (k - 2) % NBUF)
        do_chunk(j, k)
        carry = sweep_row(j, carry)
        if j + 3 < NCHUNK:
            load(j + 3, (k + 3) % NBUF)

    plsc.store_scatter(endb, [carry],
                       jnp.full((16,), RPW, jnp.int32), mask=lane0)

    def finalize(i, _):
        s16 = startb[pl.ds(i * 16, 16)]
        e16 = endb[pl.ds(i * 16, 16)]
        cntb[pl.ds(i * 16, 16)] = e16 - s16
        return 0

    lax.fori_loop(0, S // 16, finalize, 0)
    cond_wait_scat(NCHUNK - 2, (NCHUNK - 2) % NBUF)
    cond_wait_scat(NCHUNK - 1, (NCHUNK - 1) % NBUF)

    pltpu.sync_copy(cntb, pcnt_hbm.at[wid])

    plsc.subcore_barrier()

    seg0 = sid * SEG_PER_TILE
    pltpu.sync_copy(acc.at[pl.ds(seg0, SEG_PER_TILE)],
                    psum_hbm.at[cid].at[pl.ds(seg0, SEG_PER_TILE)])


_sc_call = functools.partial(
    pl.kernel,
    out_type=[
        jax.ShapeDtypeStruct((NC, S, D), jnp.float32),
        jax.ShapeDtypeStruct((NW, S), jnp.int32),
    ],
    mesh=plsc.VectorSubcoreMesh(core_axis_name="c", subcore_axis_name="s"),
    compiler_params=pltpu.CompilerParams(needs_layout_passes=False),
    scratch_types=[
        pltpu.VMEM((NCHUNK, C), jnp.int32),
        pltpu.VMEM((NCHUNK, 8), jnp.int32),
        pltpu.VMEM((C, D), jnp.float32),
        pltpu.VMEM((C, D), jnp.float32),
        pltpu.VMEM((C, D), jnp.float32),
        pltpu.VMEM((C, D), jnp.float32),
        pltpu.VMEM((C, D), jnp.float32),
        pltpu.VMEM((NBUF * 8, D), jnp.float32),
        pltpu.VMEM((SEG_PER_TILE, D), jnp.float32),
        pltpu.VMEM((16,), jnp.int32),
        pltpu.VMEM((S,), jnp.int32),
        pltpu.VMEM((S,), jnp.int32),
        pltpu.VMEM((S,), jnp.int32),
        pltpu.VMEM_SHARED((S, D), jnp.float32),
        pltpu.SemaphoreType.DMA,
        pltpu.SemaphoreType.DMA,
    ],
)(_sc_body)


def _combine_body(p_ref, c_ref, o_ref):
    sums = p_ref[0] + p_ref[1]
    cnt = jnp.sum(c_ref[...].astype(jnp.float32), axis=0)
    row_i = lax.broadcasted_iota(jnp.int32, (S, S), 0)
    col_i = lax.broadcasted_iota(jnp.int32, (S, S), 1)
    diag_cnt = jnp.where(row_i == col_i, cnt[None, :], 0.0)
    cnt_rows = jnp.dot(diag_cnt, jnp.ones((S, D), jnp.float32),
                       precision=lax.Precision.HIGHEST,
                       preferred_element_type=jnp.float32)
    o_ref[...] = sums / jnp.maximum(cnt_rows, 1.0)


_combine = pl.pallas_call(
    _combine_body,
    out_shape=jax.ShapeDtypeStruct((S, D), jnp.float32),
)


def kernel(x, batch):
    ids = batch.astype(jnp.int32).reshape(NW, NCHUNK, C)
    cfid = jnp.broadcast_to(ids[:, :, 0:1], (NW, NCHUNK, 8))
    psum, pcnt = _sc_call(x, ids, cfid)
    return _combine(psum, pcnt)
